# baseline (speedup 1.0000x reference)
import jax
import jax.numpy as jnp
from jax.experimental import pallas as pl
from jax.experimental.pallas import tpu as pltpu
from jax.experimental.pallas import tpu_sc as plsc

_LANES = 16


def _sc_add(patch2d, pos_table, num_batches):
    n, d = pos_table.shape
    rows, _ = patch2d.shape
    r = 8
    kb = 4
    n_tiles = n // r
    mesh = plsc.VectorSubcoreMesh(core_axis_name="c", subcore_axis_name="s")

    @pl.kernel(
        out_type=jax.ShapeDtypeStruct(patch2d.shape, patch2d.dtype),
        mesh=mesh,
    )
    def sc_kernel(p_hbm, t_hbm, o_hbm):
        def body(*refs):
            p_refs = refs[:kb]
            t_ref = refs[kb]
            o_refs = refs[kb + 1:]

            @plsc.parallel_loop(0, r, unroll=2)
            def _(i):
                for c in range(0, d, _LANES):
                    slc = (pl.ds(i, 1), pl.ds(c, _LANES))
                    t_val = t_ref.at[*slc][...]
                    for j in range(kb):
                        o_refs[j].at[*slc][...] = p_refs[j].at[*slc][...] + t_val

        def _pmap(j):
            return lambda b, i: ((b * kb + j) * n_tiles + i, 0)

        pltpu.emit_pipeline(
            body,
            grid=(num_batches // kb, n_tiles),
            in_specs=[pl.BlockSpec((r, d), index_map=_pmap(j)) for j in range(kb)]
            + [pl.BlockSpec((r, d), index_map=lambda b, i: (i, 0))],
            out_specs=[pl.BlockSpec((r, d), index_map=_pmap(j)) for j in range(kb)],
            core_axis_name=("c", "s"),
            dimension_semantics=(pltpu.PARALLEL, pltpu.ARBITRARY),
        )(*([p_hbm] * kb), t_hbm, *([o_hbm] * kb))

    return sc_kernel(patch2d, pos_table)


def kernel(patch, pos_table):
    batch, num_patches, proj_dim = patch.shape
    patch2d = patch.reshape(batch * num_patches, proj_dim)
    out2d = _sc_add(patch2d, pos_table, batch)
    return out2d.reshape(batch, num_patches, proj_dim)


# speedup vs baseline: 2.9506x; 2.9506x over previous
import jax
import jax.numpy as jnp
from jax.experimental import pallas as pl
from jax.experimental.pallas import tpu as pltpu
from jax.experimental.pallas import tpu_sc as plsc

_LANES = 16
_SC_BATCHES = 16


def _tc_add(patch, pos_table, b_tc):
    _, num_patches, proj_dim = patch.shape
    block_b = 8
    return pl.pallas_call(
        lambda p_ref, t_ref, o_ref: o_ref.__setitem__(
            ..., p_ref[...] + t_ref[...]
        ),
        grid=(b_tc // block_b,),
        in_specs=[
            pl.BlockSpec((block_b, num_patches, proj_dim), lambda b: (b, 0, 0)),
            pl.BlockSpec((num_patches, proj_dim), lambda b: (0, 0)),
        ],
        out_specs=pl.BlockSpec((block_b, num_patches, proj_dim), lambda b: (b, 0, 0)),
        out_shape=jax.ShapeDtypeStruct((b_tc, num_patches, proj_dim), patch.dtype),
    )(patch, pos_table)


def _sc_add(patch2d, pos_table, row_offset):
    n, d = pos_table.shape
    r = 16
    row_block_offset = row_offset // r
    n_tiles = n // r
    n_blocks = _SC_BATCHES * n // r
    mesh = plsc.VectorSubcoreMesh(core_axis_name="c", subcore_axis_name="s")

    @pl.kernel(
        out_type=jax.ShapeDtypeStruct((_SC_BATCHES * n, d), patch2d.dtype),
        mesh=mesh,
    )
    def sc_kernel(p_hbm, t_hbm, o_hbm):
        def body(p_ref, t_ref, o_ref):
            @plsc.parallel_loop(0, r, unroll=2)
            def _(i):
                for c in range(0, d, _LANES):
                    slc = (pl.ds(i, 1), pl.ds(c, _LANES))
                    o_ref.at[*slc][...] = p_ref.at[*slc][...] + t_ref.at[*slc][...]

        pltpu.emit_pipeline(
            body,
            grid=(n_blocks,),
            in_specs=[
                pl.BlockSpec((r, d), index_map=lambda k: (k + row_block_offset, 0)),
                pl.BlockSpec((r, d), index_map=lambda k: (k % n_tiles, 0)),
            ],
            out_specs=[
                pl.BlockSpec((r, d), index_map=lambda k: (k, 0)),
            ],
            core_axis_name=("c", "s"),
            dimension_semantics=(pltpu.PARALLEL,),
        )(p_hbm, t_hbm, o_hbm)

    return sc_kernel(patch2d, pos_table)


def kernel(patch, pos_table):
    batch, num_patches, proj_dim = patch.shape
    b_tc = batch - _SC_BATCHES
    patch2d = patch.reshape(batch * num_patches, proj_dim)
    sc_out = _sc_add(patch2d, pos_table, b_tc * num_patches)
    tc_out = _tc_add(patch, pos_table, b_tc)
    return jnp.concatenate(
        [tc_out, sc_out.reshape(_SC_BATCHES, num_patches, proj_dim)], axis=0
    )


# speedup vs baseline: 7.3368x; 2.4865x over previous
import jax
import jax.numpy as jnp
from jax.experimental import pallas as pl


def _add_kernel(patch_ref, table_ref, out_ref):
    out_ref[...] = patch_ref[...] + table_ref[...]


def kernel(patch, pos_table):
    batch, num_patches, proj_dim = patch.shape
    block_b = 8
    return pl.pallas_call(
        _add_kernel,
        grid=(batch // block_b,),
        in_specs=[
            pl.BlockSpec((block_b, num_patches, proj_dim), lambda b: (b, 0, 0)),
            pl.BlockSpec((num_patches, proj_dim), lambda b: (0, 0)),
        ],
        out_specs=pl.BlockSpec((block_b, num_patches, proj_dim), lambda b: (b, 0, 0)),
        out_shape=jax.ShapeDtypeStruct(patch.shape, patch.dtype),
    )(patch, pos_table)
